# baseline (speedup 1.0000x reference)
import functools

import jax
import jax.numpy as jnp
from jax import lax
from jax.experimental import pallas as pl
from jax.experimental.pallas import tpu as pltpu
from jax.experimental.pallas import tpu_sc as plsc

B = 16384
D = 16
R = 5
NC = 2
NS = 16
NW = NC * NS
CHUNK = B // NW
GROUP = 16


def _bcast_lane(v, i):
    idx = jnp.full((D,), i, dtype=jnp.int32)
    return v.at[idx].get(mode="promise_in_bounds")


def _sc_body(gu_hbm, gi_hbm, p_hbm, a0_hbm, a1_hbm, rel_hbm,
             xui_hbm, pui_hbm,
             gu_v, gi_v, p_v, a0_v, a1_v, rel_v, pui_v, xui_v):
    wid = lax.axis_index("s") * NC + lax.axis_index("c")
    base = wid * CHUNK

    pltpu.sync_copy(gu_hbm.at[pl.ds(base, CHUNK), :], gu_v)
    pltpu.sync_copy(gi_hbm.at[pl.ds(base, CHUNK), :], gi_v)
    pltpu.sync_copy(p_hbm, p_v)
    pltpu.sync_copy(a0_hbm, a0_v)
    pltpu.sync_copy(a1_hbm, a1_v)
    pltpu.sync_copy(rel_hbm, rel_v)

    p_rows = [[p_v[s, i, :] for i in range(D)] for s in range(2)]
    a0 = a0_v[...]
    a1 = a1_v[...]
    rel = rel_v[...]
    lane = lax.iota(jnp.int32, D)
    valid = lane < R
    neg_inf = jnp.float32(float("-inf"))

    def group_body(g, carry):
        del carry
        gbase = g * GROUP
        xv = jnp.zeros((D,), jnp.float32)
        for e in range(GROUP):
            row = gbase + e
            zu = gu_v[row, :]
            zi = gi_v[row, :]
            u0 = _bcast_lane(zu, 0) * p_rows[0][0]
            u1 = _bcast_lane(zu, 0) * p_rows[1][0]
            for i in range(1, D):
                bi = _bcast_lane(zu, i)
                u0 = u0 + bi * p_rows[0][i]
                u1 = u1 + bi * p_rows[1][i]
            m0 = jnp.sum(u0 * zi)
            m1 = jnp.sum(u1 * zi)
            pui = m0 * a0 + m1 * a1
            pui_v[row, :] = pui
            masked = jnp.where(valid, pui, neg_inf)
            mx = jnp.max(masked)
            ex = jnp.exp(masked - mx)
            den = jnp.sum(ex)
            num = jnp.sum(ex * rel)
            ratio = jnp.full((D,), num, jnp.float32) / jnp.full(
                (D,), den, jnp.float32)
            xv = jnp.where(lane == e, ratio, xv)
        xui_v[pl.ds(gbase, GROUP)] = xv
        return 0

    lax.fori_loop(0, CHUNK // GROUP, group_body, 0)

    pltpu.sync_copy(xui_v, xui_hbm.at[pl.ds(base, CHUNK)])
    pltpu.sync_copy(pui_v, pui_hbm.at[pl.ds(base, CHUNK), :])


@jax.jit
def _sc_call(gu, gi, P, a0, a1, relp):
    mesh = plsc.VectorSubcoreMesh(core_axis_name="c", subcore_axis_name="s")
    fn = pl.kernel(
        _sc_body,
        mesh=mesh,
        out_type=(
            jax.ShapeDtypeStruct((B,), jnp.float32),
            jax.ShapeDtypeStruct((B, D), jnp.float32),
        ),
        compiler_params=pltpu.CompilerParams(
            needs_layout_passes=False, use_tc_tiling_on_sc=False),
        scratch_types=[
            pltpu.VMEM((CHUNK, D), jnp.float32),
            pltpu.VMEM((CHUNK, D), jnp.float32),
            pltpu.VMEM((2, D, D), jnp.float32),
            pltpu.VMEM((D,), jnp.float32),
            pltpu.VMEM((D,), jnp.float32),
            pltpu.VMEM((D,), jnp.float32),
            pltpu.VMEM((CHUNK, D), jnp.float32),
            pltpu.VMEM((CHUNK,), jnp.float32),
        ],
    )
    return fn(gu, gi, P, a0, a1, relp)


def kernel(gu, gi, P, A, relations):
    gu = jnp.squeeze(gu)
    gi = jnp.squeeze(gi)
    a0 = jnp.zeros((D,), jnp.float32).at[:R].set(A[:, 0])
    a1 = jnp.zeros((D,), jnp.float32).at[:R].set(A[:, 1])
    relp = jnp.zeros((D,), jnp.float32).at[:R].set(relations)
    xui, pui_pad = _sc_call(gu, gi, P, a0, a1, relp)
    return (xui, pui_pad[:, :R])
